# baseline (speedup 1.0000x reference)
import jax
import jax.numpy as jnp
from jax.experimental import pallas as pl

_NUM_CLASSES = 805
_BLOCK_ROWS = 512


def _onehot_block(x_ref, o_ref):
    x = x_ref[...]
    iota = jax.lax.broadcasted_iota(jnp.int32, o_ref.shape, 1)
    o_ref[...] = jnp.where(x == iota, 1, 0)


def kernel(x):
    n, m = x.shape
    rows = n * m
    x2 = x.reshape(rows, 1)
    out = pl.pallas_call(
        _onehot_block,
        grid=(rows // _BLOCK_ROWS,),
        in_specs=[pl.BlockSpec((_BLOCK_ROWS, 1), lambda i: (i, 0))],
        out_specs=pl.BlockSpec((_BLOCK_ROWS, _NUM_CLASSES), lambda i: (i, 0)),
        out_shape=jax.ShapeDtypeStruct((rows, _NUM_CLASSES), jnp.int32),
    )(x2)
    return out.reshape(n, m, _NUM_CLASSES)


# speedup vs baseline: 1.6555x; 1.6555x over previous
import jax
import jax.numpy as jnp
from jax.experimental import pallas as pl

_NUM_CLASSES = 805
_BLOCK_ROWS = 64


def _onehot_block(x_ref, o_ref):
    x = x_ref[...]
    iota = jax.lax.broadcasted_iota(jnp.int32, o_ref.shape, 2)
    o_ref[...] = jnp.where(x[:, :, None] == iota, 1, 0)


def kernel(x):
    n, m = x.shape
    return pl.pallas_call(
        _onehot_block,
        grid=(n // _BLOCK_ROWS,),
        in_specs=[pl.BlockSpec((_BLOCK_ROWS, m), lambda i: (i, 0))],
        out_specs=pl.BlockSpec((_BLOCK_ROWS, m, _NUM_CLASSES),
                               lambda i: (i, 0, 0)),
        out_shape=jax.ShapeDtypeStruct((n, m, _NUM_CLASSES), jnp.int32),
    )(x)
